# baseline (speedup 1.0000x reference)
import functools
import jax
import jax.numpy as jnp
from jax import lax
from jax.experimental import pallas as pl
from jax.experimental.pallas import tpu as pltpu
from jax.experimental.pallas import tpu_sc as plsc

NC = 2
NS = 16
NW = NC * NS
CH = 128

N = 10000
E = 320000
D = 128
G = 64
NPAD = 10240
STRIPE = NPAD // NS
NCH = 2 * (-(-E // (NW * CH * 2)))
EPAD = NW * NCH * CH
NCH0 = 96
NCH1 = 64
E0 = NS * NCH0 * CH
E1 = NS * NCH1 * CH
NCHM = max(NCH0, NCH1)

_mesh = plsc.VectorSubcoreMesh(core_axis_name="c", subcore_axis_name="s",
                               num_cores=NC, num_subcores=NS)


def _wid():
    return lax.axis_index("c") * NS + lax.axis_index("s")


def _deg_body(dst_hbm, out_hbm, dst2d, ones_v, zbuf, dacc):
    cid = lax.axis_index("c")
    sid = lax.axis_index("s")
    wid = cid * NS + sid

    @pl.loop(0, CH)
    def _(i):
        ones_v[i] = jnp.ones((16,), jnp.float32)
        zbuf[i] = jnp.zeros((16,), jnp.float32)

    @pl.loop(0, STRIPE // CH)
    def _(k):
        pltpu.sync_copy(zbuf, dacc.at[pl.ds(sid * STRIPE + k * CH, CH)])
    plsc.subcore_barrier()

    pltpu.sync_copy(dst_hbm.at[wid], dst2d)

    @pl.loop(0, NCH)
    def _(j):
        pltpu.sync_copy(ones_v, dacc.at[dst2d.at[j]], add=True)
    plsc.subcore_barrier()

    pltpu.sync_copy(dacc.at[pl.ds(sid * STRIPE, STRIPE)],
                    out_hbm.at[cid, pl.ds(sid * STRIPE, STRIPE)])


_deg_kernel = functools.partial(
    pl.kernel, _deg_body,
    out_type=jax.ShapeDtypeStruct((NC, NPAD, 16), jnp.float32),
    mesh=_mesh,
    scratch_types=[
        pltpu.VMEM((NCH, CH), jnp.int32),
        pltpu.VMEM((CH, 16), jnp.float32),
        pltpu.VMEM((CH, 16), jnp.float32),
        pltpu.VMEM_SHARED((NPAD, 16), jnp.float32),
    ],
)()


def _agg_body(y_hbm, src_hbm, dst_hbm, out_hbm, src2d, dst2d, rows, acc, sem):
    cid = lax.axis_index("c")
    sid = lax.axis_index("s")
    wid = cid * NS + sid

    @pl.loop(0, CH)
    def _(i):
        for c in range(D // 16):
            rows[i, pl.ds(c * 16, 16)] = jnp.zeros((16,), jnp.float32)

    @pl.loop(0, STRIPE // CH)
    def _(k):
        pltpu.sync_copy(rows, acc.at[pl.ds(sid * STRIPE + k * CH, CH)])
    plsc.subcore_barrier()

    pltpu.sync_copy(src_hbm.at[wid], src2d)
    pltpu.sync_copy(dst_hbm.at[wid], dst2d)

    @pl.when(cid == 0)
    def _():
        @pl.loop(0, NCH0)
        def _(j):
            pltpu.async_copy(y_hbm.at[src2d.at[j]], rows, sem).wait()
            pltpu.sync_copy(rows, acc.at[dst2d.at[j]], add=True)

    @pl.when(cid == 1)
    def _():
        @pl.loop(0, NCH1)
        def _(j):
            pltpu.async_copy(y_hbm.at[src2d.at[j]], rows, sem).wait()
            pltpu.sync_copy(rows, acc.at[dst2d.at[j]], add=True)

    plsc.subcore_barrier()

    pltpu.sync_copy(acc.at[pl.ds(sid * STRIPE, STRIPE)],
                    out_hbm.at[cid, pl.ds(sid * STRIPE, STRIPE)])


_agg_kernel = functools.partial(
    pl.kernel, _agg_body,
    out_type=jax.ShapeDtypeStruct((NC, NPAD, D), jnp.float32),
    mesh=_mesh,
    scratch_types=[
        pltpu.VMEM((NCHM, CH), jnp.int32),
        pltpu.VMEM((NCHM, CH), jnp.int32),
        pltpu.VMEM((CH, D), jnp.float32),
        pltpu.VMEM_SHARED((NPAD, D), jnp.float32),
        pltpu.SemaphoreType.DMA,
    ],
)()


BLK = 1024
GRID = NPAD // BLK


def _dinv_from_deg(degp_blk):
    deg = degp_blk[0, :, 0:1] + degp_blk[1, :, 0:1] + 1.0
    return lax.rsqrt(deg)


def _y1_body(x_ref, w_ref, degp_ref, y_ref):
    dinv = _dinv_from_deg(degp_ref)
    xw = lax.dot_general(x_ref[...], w_ref[...], (((1,), (1,)), ((), ())),
                         preferred_element_type=jnp.float32)
    y_ref[...] = dinv * xw


def _y1_call(x, w1, degp):
    return pl.pallas_call(
        _y1_body,
        grid=(GRID,),
        in_specs=[
            pl.BlockSpec((BLK, D), lambda j: (j, 0)),
            pl.BlockSpec((D, D), lambda j: (0, 0)),
            pl.BlockSpec((NC, BLK, 16), lambda j: (0, j, 0)),
        ],
        out_specs=pl.BlockSpec((BLK, D), lambda j: (j, 0)),
        out_shape=jax.ShapeDtypeStruct((NPAD, D), jnp.float32),
    )(x, w1, degp)


def _combine_body(acc_ref, y_ref, degp_ref, b_ref, w_ref,
                  conv_ref, h_ref, ynext_ref):
    dinv = _dinv_from_deg(degp_ref)
    conv = dinv * (acc_ref[0] + acc_ref[1] + y_ref[...]) + b_ref[...]
    conv_ref[...] = conv
    h = jnp.maximum(conv, 0.0)
    h_ref[...] = h
    hw = lax.dot_general(h, w_ref[...], (((1,), (1,)), ((), ())),
                         preferred_element_type=jnp.float32)
    ynext_ref[...] = dinv * hw


def _combine_call(acc, y, degp, b, w_next):
    return pl.pallas_call(
        _combine_body,
        grid=(GRID,),
        in_specs=[
            pl.BlockSpec((NC, BLK, D), lambda j: (0, j, 0)),
            pl.BlockSpec((BLK, D), lambda j: (j, 0)),
            pl.BlockSpec((NC, BLK, 16), lambda j: (0, j, 0)),
            pl.BlockSpec((1, D), lambda j: (0, 0)),
            pl.BlockSpec((D, D), lambda j: (0, 0)),
        ],
        out_specs=[
            pl.BlockSpec((BLK, D), lambda j: (j, 0)),
            pl.BlockSpec((BLK, D), lambda j: (j, 0)),
            pl.BlockSpec((BLK, D), lambda j: (j, 0)),
        ],
        out_shape=[
            jax.ShapeDtypeStruct((NPAD, D), jnp.float32),
            jax.ShapeDtypeStruct((NPAD, D), jnp.float32),
            jax.ShapeDtypeStruct((NPAD, D), jnp.float32),
        ],
    )(acc, y, degp, b, w_next)


def _pool_body(acc_ref, y_ref, degp_ref, b_ref, batch_ref, wfc_ref, bfc_ref,
               conv_ref, pooled_ref, fc_ref, psum, pcnt):
    j = pl.program_id(0)
    dinv = _dinv_from_deg(degp_ref)
    conv = dinv * (acc_ref[0] + acc_ref[1] + y_ref[...]) + b_ref[...]
    conv_ref[...] = conv

    seg = lax.broadcasted_iota(jnp.int32, (1, G), 1)
    p = (batch_ref[...] == seg).astype(jnp.float32)
    part = lax.dot_general(p, conv, (((0,), (0,)), ((), ())),
                           preferred_element_type=jnp.float32)
    ones = jnp.ones((BLK, D), jnp.float32)
    cpart = lax.dot_general(p, ones, (((0,), (0,)), ((), ())),
                            preferred_element_type=jnp.float32)

    @pl.when(j == 0)
    def _():
        psum[...] = jnp.zeros((G, D), jnp.float32)
        pcnt[...] = jnp.zeros((G, D), jnp.float32)

    psum[...] += part
    pcnt[...] += cpart

    @pl.when(j == GRID - 1)
    def _():
        pooled = psum[...] / jnp.maximum(pcnt[...], 1.0)
        pooled_ref[...] = pooled
        fc = lax.dot_general(pooled, wfc_ref[...], (((1,), (1,)), ((), ())),
                             preferred_element_type=jnp.float32)
        fc_ref[...] = fc + bfc_ref[...]


def _pool_call(acc, y, degp, b, batch2d, wfc, bfc):
    ncls = wfc.shape[0]
    return pl.pallas_call(
        _pool_body,
        grid=(GRID,),
        in_specs=[
            pl.BlockSpec((NC, BLK, D), lambda j: (0, j, 0)),
            pl.BlockSpec((BLK, D), lambda j: (j, 0)),
            pl.BlockSpec((NC, BLK, 16), lambda j: (0, j, 0)),
            pl.BlockSpec((1, D), lambda j: (0, 0)),
            pl.BlockSpec((BLK, 1), lambda j: (j, 0)),
            pl.BlockSpec((ncls, D), lambda j: (0, 0)),
            pl.BlockSpec((1, ncls), lambda j: (0, 0)),
        ],
        out_specs=[
            pl.BlockSpec((BLK, D), lambda j: (j, 0)),
            pl.BlockSpec((G, D), lambda j: (0, 0)),
            pl.BlockSpec((G, ncls), lambda j: (0, 0)),
        ],
        out_shape=[
            jax.ShapeDtypeStruct((NPAD, D), jnp.float32),
            jax.ShapeDtypeStruct((G, D), jnp.float32),
            jax.ShapeDtypeStruct((G, ncls), jnp.float32),
        ],
        scratch_shapes=[
            pltpu.VMEM((G, D), jnp.float32),
            pltpu.VMEM((G, D), jnp.float32),
        ],
    )(acc, y, degp, b, batch2d, wfc, bfc)


def kernel(x, edge_index, batch, W1, b1, W2, b2, W3, b3, Wfc, bfc):
    src = edge_index[0].astype(jnp.int32)
    dst = edge_index[1].astype(jnp.int32)
    def split_uneven(a, fill):
        pad = jnp.full((E0 + E1 - E,), fill, jnp.int32)
        ap = jnp.concatenate([a, pad])
        a0 = ap[:E0].reshape(NS, NCH0, CH)
        a1 = ap[E0:].reshape(NS, NCH1, CH)
        tail = jnp.full((NS, NCHM - NCH1, CH), fill, jnp.int32)
        a1 = jnp.concatenate([a1, tail], axis=1)
        tail0 = jnp.full((NS, NCHM - NCH0, CH), fill, jnp.int32)
        a0 = jnp.concatenate([a0, tail0], axis=1)
        return jnp.concatenate([a0, a1], axis=0)

    src_p = split_uneven(src, 0)
    dst_p = split_uneven(dst, N)
    dst_deg = jnp.concatenate([dst, jnp.full((EPAD - E,), N, jnp.int32)])
    dst_deg = dst_deg.reshape(NW, NCH, CH)

    x_p = jnp.pad(x, ((0, NPAD - N), (0, 0)))
    batch_p = jnp.pad(batch.astype(jnp.int32), (0, NPAD - N),
                      constant_values=G).reshape(NPAD, 1)

    degp = _deg_kernel(dst_deg)

    y1 = _y1_call(x_p, W1, degp)
    acc1 = _agg_kernel(y1, src_p, dst_p)
    conv1, relu1, y2 = _combine_call(acc1, y1, degp, b1.reshape(1, D), W2)

    acc2 = _agg_kernel(y2, src_p, dst_p)
    conv2, _relu2, y3 = _combine_call(acc2, y2, degp, b2.reshape(1, D), W3)

    acc3 = _agg_kernel(y3, src_p, dst_p)
    conv3, pooled, fc = _pool_call(acc3, y3, degp, b3.reshape(1, D),
                                   batch_p, Wfc, bfc.reshape(1, -1))

    activations = {
        "conv1": conv1[:N],
        "relu1": relu1[:N],
        "conv2": conv2[:N],
        "conv3": conv3[:N],
        "global_pool": pooled,
        "fc": fc,
    }
    return (fc, activations)


# speedup vs baseline: 1.0126x; 1.0126x over previous
import functools
import jax
import jax.numpy as jnp
from jax import lax
from jax.experimental import pallas as pl
from jax.experimental.pallas import tpu as pltpu
from jax.experimental.pallas import tpu_sc as plsc

NC = 2
NS = 16
NW = NC * NS
CH = 128

N = 10000
E = 320000
D = 128
G = 64
NPAD = 10240
STRIPE = NPAD // NS
NCH = 2 * (-(-E // (NW * CH * 2)))
EPAD = NW * NCH * CH
NCH0 = 64
NCH1 = 96
E0 = NS * NCH0 * CH
E1 = NS * NCH1 * CH
NCHM = max(NCH0, NCH1)

_mesh = plsc.VectorSubcoreMesh(core_axis_name="c", subcore_axis_name="s",
                               num_cores=NC, num_subcores=NS)


def _wid():
    return lax.axis_index("c") * NS + lax.axis_index("s")


def _deg_body(dst_hbm, out_hbm, dst2d, ones_v, zbuf, dacc):
    cid = lax.axis_index("c")
    sid = lax.axis_index("s")
    wid = cid * NS + sid

    @pl.loop(0, CH)
    def _(i):
        ones_v[i] = jnp.ones((16,), jnp.float32)
        zbuf[i] = jnp.zeros((16,), jnp.float32)

    @pl.loop(0, STRIPE // CH)
    def _(k):
        pltpu.sync_copy(zbuf, dacc.at[pl.ds(sid * STRIPE + k * CH, CH)])
    plsc.subcore_barrier()

    pltpu.sync_copy(dst_hbm.at[wid], dst2d)

    @pl.loop(0, NCH)
    def _(j):
        pltpu.sync_copy(ones_v, dacc.at[dst2d.at[j]], add=True)
    plsc.subcore_barrier()

    pltpu.sync_copy(dacc.at[pl.ds(sid * STRIPE, STRIPE)],
                    out_hbm.at[cid, pl.ds(sid * STRIPE, STRIPE)])


_deg_kernel = functools.partial(
    pl.kernel, _deg_body,
    out_type=jax.ShapeDtypeStruct((NC, NPAD, 16), jnp.float32),
    mesh=_mesh,
    scratch_types=[
        pltpu.VMEM((NCH, CH), jnp.int32),
        pltpu.VMEM((CH, 16), jnp.float32),
        pltpu.VMEM((CH, 16), jnp.float32),
        pltpu.VMEM_SHARED((NPAD, 16), jnp.float32),
    ],
)()


def _agg_body(y_hbm, src_hbm, dst_hbm, out_hbm, src2d, dst2d, rows, acc, sem):
    cid = lax.axis_index("c")
    sid = lax.axis_index("s")
    wid = cid * NS + sid

    @pl.loop(0, CH)
    def _(i):
        for c in range(D // 16):
            rows[i, pl.ds(c * 16, 16)] = jnp.zeros((16,), jnp.float32)

    @pl.loop(0, STRIPE // CH)
    def _(k):
        pltpu.sync_copy(rows, acc.at[pl.ds(sid * STRIPE + k * CH, CH)])
    plsc.subcore_barrier()

    pltpu.sync_copy(src_hbm.at[wid], src2d)
    pltpu.sync_copy(dst_hbm.at[wid], dst2d)

    @pl.when(cid == 0)
    def _():
        @pl.loop(0, NCH0)
        def _(j):
            pltpu.async_copy(y_hbm.at[src2d.at[j]], rows, sem).wait()
            pltpu.sync_copy(rows, acc.at[dst2d.at[j]], add=True)

    @pl.when(cid == 1)
    def _():
        @pl.loop(0, NCH1)
        def _(j):
            pltpu.async_copy(y_hbm.at[src2d.at[j]], rows, sem).wait()
            pltpu.sync_copy(rows, acc.at[dst2d.at[j]], add=True)

    plsc.subcore_barrier()

    pltpu.sync_copy(acc.at[pl.ds(sid * STRIPE, STRIPE)],
                    out_hbm.at[cid, pl.ds(sid * STRIPE, STRIPE)])


_agg_kernel = functools.partial(
    pl.kernel, _agg_body,
    out_type=jax.ShapeDtypeStruct((NC, NPAD, D), jnp.float32),
    mesh=_mesh,
    scratch_types=[
        pltpu.VMEM((NCHM, CH), jnp.int32),
        pltpu.VMEM((NCHM, CH), jnp.int32),
        pltpu.VMEM((CH, D), jnp.float32),
        pltpu.VMEM_SHARED((NPAD, D), jnp.float32),
        pltpu.SemaphoreType.DMA,
    ],
)()


BLK = 1024
GRID = NPAD // BLK


def _dinv_from_deg(degp_blk):
    deg = degp_blk[0, :, 0:1] + degp_blk[1, :, 0:1] + 1.0
    return lax.rsqrt(deg)


def _y1_body(x_ref, w_ref, degp_ref, y_ref):
    dinv = _dinv_from_deg(degp_ref)
    xw = lax.dot_general(x_ref[...], w_ref[...], (((1,), (1,)), ((), ())),
                         preferred_element_type=jnp.float32)
    y_ref[...] = dinv * xw


def _y1_call(x, w1, degp):
    return pl.pallas_call(
        _y1_body,
        grid=(GRID,),
        in_specs=[
            pl.BlockSpec((BLK, D), lambda j: (j, 0)),
            pl.BlockSpec((D, D), lambda j: (0, 0)),
            pl.BlockSpec((NC, BLK, 16), lambda j: (0, j, 0)),
        ],
        out_specs=pl.BlockSpec((BLK, D), lambda j: (j, 0)),
        out_shape=jax.ShapeDtypeStruct((NPAD, D), jnp.float32),
    )(x, w1, degp)


def _combine_body(acc_ref, y_ref, degp_ref, b_ref, w_ref,
                  conv_ref, h_ref, ynext_ref):
    dinv = _dinv_from_deg(degp_ref)
    conv = dinv * (acc_ref[0] + acc_ref[1] + y_ref[...]) + b_ref[...]
    conv_ref[...] = conv
    h = jnp.maximum(conv, 0.0)
    h_ref[...] = h
    hw = lax.dot_general(h, w_ref[...], (((1,), (1,)), ((), ())),
                         preferred_element_type=jnp.float32)
    ynext_ref[...] = dinv * hw


def _combine_call(acc, y, degp, b, w_next):
    return pl.pallas_call(
        _combine_body,
        grid=(GRID,),
        in_specs=[
            pl.BlockSpec((NC, BLK, D), lambda j: (0, j, 0)),
            pl.BlockSpec((BLK, D), lambda j: (j, 0)),
            pl.BlockSpec((NC, BLK, 16), lambda j: (0, j, 0)),
            pl.BlockSpec((1, D), lambda j: (0, 0)),
            pl.BlockSpec((D, D), lambda j: (0, 0)),
        ],
        out_specs=[
            pl.BlockSpec((BLK, D), lambda j: (j, 0)),
            pl.BlockSpec((BLK, D), lambda j: (j, 0)),
            pl.BlockSpec((BLK, D), lambda j: (j, 0)),
        ],
        out_shape=[
            jax.ShapeDtypeStruct((NPAD, D), jnp.float32),
            jax.ShapeDtypeStruct((NPAD, D), jnp.float32),
            jax.ShapeDtypeStruct((NPAD, D), jnp.float32),
        ],
    )(acc, y, degp, b, w_next)


def _pool_body(acc_ref, y_ref, degp_ref, b_ref, batch_ref, wfc_ref, bfc_ref,
               conv_ref, pooled_ref, fc_ref, psum, pcnt):
    j = pl.program_id(0)
    dinv = _dinv_from_deg(degp_ref)
    conv = dinv * (acc_ref[0] + acc_ref[1] + y_ref[...]) + b_ref[...]
    conv_ref[...] = conv

    seg = lax.broadcasted_iota(jnp.int32, (1, G), 1)
    p = (batch_ref[...] == seg).astype(jnp.float32)
    part = lax.dot_general(p, conv, (((0,), (0,)), ((), ())),
                           preferred_element_type=jnp.float32)
    ones = jnp.ones((BLK, D), jnp.float32)
    cpart = lax.dot_general(p, ones, (((0,), (0,)), ((), ())),
                            preferred_element_type=jnp.float32)

    @pl.when(j == 0)
    def _():
        psum[...] = jnp.zeros((G, D), jnp.float32)
        pcnt[...] = jnp.zeros((G, D), jnp.float32)

    psum[...] += part
    pcnt[...] += cpart

    @pl.when(j == GRID - 1)
    def _():
        pooled = psum[...] / jnp.maximum(pcnt[...], 1.0)
        pooled_ref[...] = pooled
        fc = lax.dot_general(pooled, wfc_ref[...], (((1,), (1,)), ((), ())),
                             preferred_element_type=jnp.float32)
        fc_ref[...] = fc + bfc_ref[...]


def _pool_call(acc, y, degp, b, batch2d, wfc, bfc):
    ncls = wfc.shape[0]
    return pl.pallas_call(
        _pool_body,
        grid=(GRID,),
        in_specs=[
            pl.BlockSpec((NC, BLK, D), lambda j: (0, j, 0)),
            pl.BlockSpec((BLK, D), lambda j: (j, 0)),
            pl.BlockSpec((NC, BLK, 16), lambda j: (0, j, 0)),
            pl.BlockSpec((1, D), lambda j: (0, 0)),
            pl.BlockSpec((BLK, 1), lambda j: (j, 0)),
            pl.BlockSpec((ncls, D), lambda j: (0, 0)),
            pl.BlockSpec((1, ncls), lambda j: (0, 0)),
        ],
        out_specs=[
            pl.BlockSpec((BLK, D), lambda j: (j, 0)),
            pl.BlockSpec((G, D), lambda j: (0, 0)),
            pl.BlockSpec((G, ncls), lambda j: (0, 0)),
        ],
        out_shape=[
            jax.ShapeDtypeStruct((NPAD, D), jnp.float32),
            jax.ShapeDtypeStruct((G, D), jnp.float32),
            jax.ShapeDtypeStruct((G, ncls), jnp.float32),
        ],
        scratch_shapes=[
            pltpu.VMEM((G, D), jnp.float32),
            pltpu.VMEM((G, D), jnp.float32),
        ],
    )(acc, y, degp, b, batch2d, wfc, bfc)


def kernel(x, edge_index, batch, W1, b1, W2, b2, W3, b3, Wfc, bfc):
    src = edge_index[0].astype(jnp.int32)
    dst = edge_index[1].astype(jnp.int32)
    def split_uneven(a, fill):
        pad = jnp.full((E0 + E1 - E,), fill, jnp.int32)
        ap = jnp.concatenate([a, pad])
        a0 = ap[:E0].reshape(NS, NCH0, CH)
        a1 = ap[E0:].reshape(NS, NCH1, CH)
        t0 = jnp.full((NS, NCHM - NCH0, CH), fill, jnp.int32)
        t1 = jnp.full((NS, NCHM - NCH1, CH), fill, jnp.int32)
        a0 = jnp.concatenate([a0, t0], axis=1)
        a1 = jnp.concatenate([a1, t1], axis=1)
        return jnp.concatenate([a0, a1], axis=0)

    src_p = split_uneven(src, 0)
    dst_p = split_uneven(dst, N)
    dst_deg = jnp.concatenate([dst, jnp.full((EPAD - E,), N, jnp.int32)])
    dst_deg = dst_deg.reshape(NW, NCH, CH)

    x_p = jnp.pad(x, ((0, NPAD - N), (0, 0)))
    batch_p = jnp.pad(batch.astype(jnp.int32), (0, NPAD - N),
                      constant_values=G).reshape(NPAD, 1)

    degp = _deg_kernel(dst_deg)

    y1 = _y1_call(x_p, W1, degp)
    acc1 = _agg_kernel(y1, src_p, dst_p)
    conv1, relu1, y2 = _combine_call(acc1, y1, degp, b1.reshape(1, D), W2)

    acc2 = _agg_kernel(y2, src_p, dst_p)
    conv2, _relu2, y3 = _combine_call(acc2, y2, degp, b2.reshape(1, D), W3)

    acc3 = _agg_kernel(y3, src_p, dst_p)
    conv3, pooled, fc = _pool_call(acc3, y3, degp, b3.reshape(1, D),
                                   batch_p, Wfc, bfc.reshape(1, -1))

    activations = {
        "conv1": conv1[:N],
        "relu1": relu1[:N],
        "conv2": conv2[:N],
        "conv3": conv3[:N],
        "global_pool": pooled,
        "fc": fc,
    }
    return (fc, activations)


# speedup vs baseline: 1.4078x; 1.3903x over previous
import functools
import jax
import jax.numpy as jnp
from jax import lax
from jax.experimental import pallas as pl
from jax.experimental.pallas import tpu as pltpu
from jax.experimental.pallas import tpu_sc as plsc

NC = 2
NS = 16
NW = NC * NS
CH = 128

N = 10000
E = 320000
D = 128
G = 64
NPAD = 10240
STRIPE = NPAD // NS
NCH = -(-E // (NW * CH))
EPAD = NW * NCH * CH

_mesh = plsc.VectorSubcoreMesh(core_axis_name="c", subcore_axis_name="s",
                               num_cores=NC, num_subcores=NS)


def _wid():
    return lax.axis_index("c") * NS + lax.axis_index("s")


def _deg_body(dst_hbm, out_hbm, dst2d, ones_v, zbuf, dacc):
    cid = lax.axis_index("c")
    sid = lax.axis_index("s")
    wid = cid * NS + sid

    @pl.loop(0, CH)
    def _(i):
        ones_v[i] = jnp.ones((16,), jnp.float32)
        zbuf[i] = jnp.zeros((16,), jnp.float32)

    @pl.loop(0, STRIPE // CH)
    def _(k):
        pltpu.sync_copy(zbuf, dacc.at[pl.ds(sid * STRIPE + k * CH, CH)])
    plsc.subcore_barrier()

    pltpu.sync_copy(dst_hbm.at[wid], dst2d)

    @pl.loop(0, NCH)
    def _(j):
        pltpu.sync_copy(ones_v, dacc.at[dst2d.at[j]], add=True)
    plsc.subcore_barrier()

    pltpu.sync_copy(dacc.at[pl.ds(sid * STRIPE, STRIPE)],
                    out_hbm.at[cid, pl.ds(sid * STRIPE, STRIPE)])


_deg_kernel = functools.partial(
    pl.kernel, _deg_body,
    out_type=jax.ShapeDtypeStruct((NC, NPAD, 16), jnp.float32),
    mesh=_mesh,
    scratch_types=[
        pltpu.VMEM((NCH, CH), jnp.int32),
        pltpu.VMEM((CH, 16), jnp.float32),
        pltpu.VMEM((CH, 16), jnp.float32),
        pltpu.VMEM_SHARED((NPAD, 16), jnp.float32),
    ],
)()


def _agg_body(y_hbm, src_hbm, dst_hbm, out_hbm, src2d, dst2d, rows, acc, sem):
    cid = lax.axis_index("c")
    sid = lax.axis_index("s")
    wid = cid * NS + sid

    @pl.loop(0, CH)
    def _(i):
        for c in range(D // 16):
            rows[i, pl.ds(c * 16, 16)] = jnp.zeros((16,), jnp.float32)

    @pl.loop(0, STRIPE // CH)
    def _(k):
        pltpu.sync_copy(rows, acc.at[pl.ds(sid * STRIPE + k * CH, CH)])
    plsc.subcore_barrier()

    pltpu.sync_copy(src_hbm.at[wid], src2d)
    pltpu.sync_copy(dst_hbm.at[wid], dst2d)

    @pl.loop(0, NCH)
    def _(j):
        pltpu.async_copy(y_hbm.at[src2d.at[j]], rows, sem).wait()
        pltpu.sync_copy(rows, acc.at[dst2d.at[j]], add=True)
    plsc.subcore_barrier()

    pltpu.sync_copy(acc.at[pl.ds(sid * STRIPE, STRIPE)],
                    out_hbm.at[cid, pl.ds(sid * STRIPE, STRIPE)])


_agg_kernel = functools.partial(
    pl.kernel, _agg_body,
    out_type=jax.ShapeDtypeStruct((NC, NPAD, D), jnp.float32),
    mesh=_mesh,
    scratch_types=[
        pltpu.VMEM((NCH, CH), jnp.int32),
        pltpu.VMEM((NCH, CH), jnp.int32),
        pltpu.VMEM((CH, D), jnp.float32),
        pltpu.VMEM_SHARED((NPAD, D), jnp.float32),
        pltpu.SemaphoreType.DMA,
    ],
)()


BLK = 1024
GRID = NPAD // BLK


def _dinv_from_deg(degp_blk):
    deg = degp_blk[0, :, 0:1] + degp_blk[1, :, 0:1] + 1.0
    return lax.rsqrt(deg)


def _y1_body(x_ref, w_ref, degp_ref, y_ref):
    dinv = _dinv_from_deg(degp_ref)
    xw = lax.dot_general(x_ref[...], w_ref[...], (((1,), (1,)), ((), ())),
                         preferred_element_type=jnp.float32)
    y_ref[...] = dinv * xw


def _y1_call(x, w1, degp):
    return pl.pallas_call(
        _y1_body,
        grid=(GRID,),
        in_specs=[
            pl.BlockSpec((BLK, D), lambda j: (j, 0)),
            pl.BlockSpec((D, D), lambda j: (0, 0)),
            pl.BlockSpec((NC, BLK, 16), lambda j: (0, j, 0)),
        ],
        out_specs=pl.BlockSpec((BLK, D), lambda j: (j, 0)),
        out_shape=jax.ShapeDtypeStruct((NPAD, D), jnp.float32),
    )(x, w1, degp)


def _combine_body(acc_ref, y_ref, degp_ref, b_ref, w_ref,
                  conv_ref, h_ref, ynext_ref):
    dinv = _dinv_from_deg(degp_ref)
    conv = dinv * (acc_ref[0] + acc_ref[1] + y_ref[...]) + b_ref[...]
    conv_ref[...] = conv
    h = jnp.maximum(conv, 0.0)
    h_ref[...] = h
    hw = lax.dot_general(h, w_ref[...], (((1,), (1,)), ((), ())),
                         preferred_element_type=jnp.float32)
    ynext_ref[...] = dinv * hw


def _combine_call(acc, y, degp, b, w_next):
    return pl.pallas_call(
        _combine_body,
        grid=(GRID,),
        in_specs=[
            pl.BlockSpec((NC, BLK, D), lambda j: (0, j, 0)),
            pl.BlockSpec((BLK, D), lambda j: (j, 0)),
            pl.BlockSpec((NC, BLK, 16), lambda j: (0, j, 0)),
            pl.BlockSpec((1, D), lambda j: (0, 0)),
            pl.BlockSpec((D, D), lambda j: (0, 0)),
        ],
        out_specs=[
            pl.BlockSpec((BLK, D), lambda j: (j, 0)),
            pl.BlockSpec((BLK, D), lambda j: (j, 0)),
            pl.BlockSpec((BLK, D), lambda j: (j, 0)),
        ],
        out_shape=[
            jax.ShapeDtypeStruct((NPAD, D), jnp.float32),
            jax.ShapeDtypeStruct((NPAD, D), jnp.float32),
            jax.ShapeDtypeStruct((NPAD, D), jnp.float32),
        ],
    )(acc, y, degp, b, w_next)


def _pool_body(acc_ref, y_ref, degp_ref, b_ref, batch_ref, wfc_ref, bfc_ref,
               conv_ref, pooled_ref, fc_ref, psum, pcnt):
    j = pl.program_id(0)
    dinv = _dinv_from_deg(degp_ref)
    conv = dinv * (acc_ref[0] + acc_ref[1] + y_ref[...]) + b_ref[...]
    conv_ref[...] = conv

    seg = lax.broadcasted_iota(jnp.int32, (1, G), 1)
    p = (batch_ref[...] == seg).astype(jnp.float32)
    part = lax.dot_general(p, conv, (((0,), (0,)), ((), ())),
                           preferred_element_type=jnp.float32)
    ones = jnp.ones((BLK, D), jnp.float32)
    cpart = lax.dot_general(p, ones, (((0,), (0,)), ((), ())),
                            preferred_element_type=jnp.float32)

    @pl.when(j == 0)
    def _():
        psum[...] = jnp.zeros((G, D), jnp.float32)
        pcnt[...] = jnp.zeros((G, D), jnp.float32)

    psum[...] += part
    pcnt[...] += cpart

    @pl.when(j == GRID - 1)
    def _():
        pooled = psum[...] / jnp.maximum(pcnt[...], 1.0)
        pooled_ref[...] = pooled
        fc = lax.dot_general(pooled, wfc_ref[...], (((1,), (1,)), ((), ())),
                             preferred_element_type=jnp.float32)
        fc_ref[...] = fc + bfc_ref[...]


def _pool_call(acc, y, degp, b, batch2d, wfc, bfc):
    ncls = wfc.shape[0]
    return pl.pallas_call(
        _pool_body,
        grid=(GRID,),
        in_specs=[
            pl.BlockSpec((NC, BLK, D), lambda j: (0, j, 0)),
            pl.BlockSpec((BLK, D), lambda j: (j, 0)),
            pl.BlockSpec((NC, BLK, 16), lambda j: (0, j, 0)),
            pl.BlockSpec((1, D), lambda j: (0, 0)),
            pl.BlockSpec((BLK, 1), lambda j: (j, 0)),
            pl.BlockSpec((ncls, D), lambda j: (0, 0)),
            pl.BlockSpec((1, ncls), lambda j: (0, 0)),
        ],
        out_specs=[
            pl.BlockSpec((BLK, D), lambda j: (j, 0)),
            pl.BlockSpec((G, D), lambda j: (0, 0)),
            pl.BlockSpec((G, ncls), lambda j: (0, 0)),
        ],
        out_shape=[
            jax.ShapeDtypeStruct((NPAD, D), jnp.float32),
            jax.ShapeDtypeStruct((G, D), jnp.float32),
            jax.ShapeDtypeStruct((G, ncls), jnp.float32),
        ],
        scratch_shapes=[
            pltpu.VMEM((G, D), jnp.float32),
            pltpu.VMEM((G, D), jnp.float32),
        ],
    )(acc, y, degp, b, batch2d, wfc, bfc)


def kernel(x, edge_index, batch, W1, b1, W2, b2, W3, b3, Wfc, bfc):
    src = edge_index[0].astype(jnp.int32)
    dst = edge_index[1].astype(jnp.int32)
    src_p = jnp.concatenate([src, jnp.zeros((EPAD - E,), jnp.int32)])
    dst_p = jnp.concatenate([dst, jnp.full((EPAD - E,), N, jnp.int32)])
    src_p = src_p.reshape(NW, NCH, CH)
    dst_p = dst_p.reshape(NW, NCH, CH)

    x_p = jnp.pad(x, ((0, NPAD - N), (0, 0)))
    batch_p = jnp.pad(batch.astype(jnp.int32), (0, NPAD - N),
                      constant_values=G).reshape(NPAD, 1)

    degp = _deg_kernel(dst_p)

    y1 = _y1_call(x_p, W1, degp)
    acc1 = _agg_kernel(y1, src_p, dst_p)
    conv1, relu1, y2 = _combine_call(acc1, y1, degp, b1.reshape(1, D), W2)

    acc2 = _agg_kernel(y2, src_p, dst_p)
    conv2, _relu2, y3 = _combine_call(acc2, y2, degp, b2.reshape(1, D), W3)

    acc3 = _agg_kernel(y3, src_p, dst_p)
    conv3, pooled, fc = _pool_call(acc3, y3, degp, b3.reshape(1, D),
                                   batch_p, Wfc, bfc.reshape(1, -1))

    activations = {
        "conv1": conv1[:N],
        "relu1": relu1[:N],
        "conv2": conv2[:N],
        "conv3": conv3[:N],
        "global_pool": pooled,
        "fc": fc,
    }
    return (fc, activations)


# speedup vs baseline: 1.6855x; 1.1973x over previous
import functools
import jax
import jax.numpy as jnp
from jax import lax
from jax.experimental import pallas as pl
from jax.experimental.pallas import tpu as pltpu
from jax.experimental.pallas import tpu_sc as plsc

NC = 2
NS = 16
NW = NC * NS
CH = 64

N = 10000
E = 320000
D = 128
G = 64
NPAD = 10240
STRIPE = NPAD // NS
NCH = -(-E // (NW * CH))
EPAD = NW * NCH * CH

_mesh = plsc.VectorSubcoreMesh(core_axis_name="c", subcore_axis_name="s",
                               num_cores=NC, num_subcores=NS)


def _wid():
    return lax.axis_index("c") * NS + lax.axis_index("s")


def _deg_body(dst_hbm, out_hbm, dst2d, ones_v, zbuf, dacc):
    cid = lax.axis_index("c")
    sid = lax.axis_index("s")
    wid = cid * NS + sid

    @pl.loop(0, CH)
    def _(i):
        ones_v[i] = jnp.ones((16,), jnp.float32)
        zbuf[i] = jnp.zeros((16,), jnp.float32)

    @pl.loop(0, STRIPE // CH)
    def _(k):
        pltpu.sync_copy(zbuf, dacc.at[pl.ds(sid * STRIPE + k * CH, CH)])
    plsc.subcore_barrier()

    pltpu.sync_copy(dst_hbm.at[wid], dst2d)

    @pl.loop(0, NCH)
    def _(j):
        pltpu.sync_copy(ones_v, dacc.at[dst2d.at[j]], add=True)
    plsc.subcore_barrier()

    pltpu.sync_copy(dacc.at[pl.ds(sid * STRIPE, STRIPE)],
                    out_hbm.at[cid, pl.ds(sid * STRIPE, STRIPE)])


_deg_kernel = functools.partial(
    pl.kernel, _deg_body,
    out_type=jax.ShapeDtypeStruct((NC, NPAD, 16), jnp.float32),
    mesh=_mesh,
    scratch_types=[
        pltpu.VMEM((NCH, CH), jnp.int32),
        pltpu.VMEM((CH, 16), jnp.float32),
        pltpu.VMEM((CH, 16), jnp.float32),
        pltpu.VMEM_SHARED((NPAD, 16), jnp.float32),
    ],
)()


def _agg_body(y_hbm, src_hbm, dst_hbm, out_hbm, src2d, dst2d, rows, acc, sem):
    cid = lax.axis_index("c")
    sid = lax.axis_index("s")
    wid = cid * NS + sid

    @pl.loop(0, CH)
    def _(i):
        for c in range(D // 16):
            rows[i, pl.ds(c * 16, 16)] = jnp.zeros((16,), jnp.float32)

    @pl.loop(0, STRIPE // CH)
    def _(k):
        pltpu.sync_copy(rows, acc.at[pl.ds(sid * STRIPE + k * CH, CH)])
    plsc.subcore_barrier()

    pltpu.sync_copy(src_hbm.at[wid], src2d)
    pltpu.sync_copy(dst_hbm.at[wid], dst2d)

    @pl.loop(0, NCH)
    def _(j):
        pltpu.async_copy(y_hbm.at[src2d.at[j]], rows, sem).wait()
        pltpu.sync_copy(rows, acc.at[dst2d.at[j]], add=True)
    plsc.subcore_barrier()

    pltpu.sync_copy(acc.at[pl.ds(sid * STRIPE, STRIPE)],
                    out_hbm.at[cid, pl.ds(sid * STRIPE, STRIPE)])


_agg_kernel = functools.partial(
    pl.kernel, _agg_body,
    out_type=jax.ShapeDtypeStruct((NC, NPAD, D), jnp.float32),
    mesh=_mesh,
    scratch_types=[
        pltpu.VMEM((NCH, CH), jnp.int32),
        pltpu.VMEM((NCH, CH), jnp.int32),
        pltpu.VMEM((CH, D), jnp.float32),
        pltpu.VMEM_SHARED((NPAD, D), jnp.float32),
        pltpu.SemaphoreType.DMA,
    ],
)()


BLK = 1024
GRID = NPAD // BLK


def _dinv_from_deg(degp_blk):
    deg = degp_blk[0, :, 0:1] + degp_blk[1, :, 0:1] + 1.0
    return lax.rsqrt(deg)


def _y1_body(x_ref, w_ref, degp_ref, y_ref):
    dinv = _dinv_from_deg(degp_ref)
    xw = lax.dot_general(x_ref[...], w_ref[...], (((1,), (1,)), ((), ())),
                         preferred_element_type=jnp.float32)
    y_ref[...] = dinv * xw


def _y1_call(x, w1, degp):
    return pl.pallas_call(
        _y1_body,
        grid=(GRID,),
        in_specs=[
            pl.BlockSpec((BLK, D), lambda j: (j, 0)),
            pl.BlockSpec((D, D), lambda j: (0, 0)),
            pl.BlockSpec((NC, BLK, 16), lambda j: (0, j, 0)),
        ],
        out_specs=pl.BlockSpec((BLK, D), lambda j: (j, 0)),
        out_shape=jax.ShapeDtypeStruct((NPAD, D), jnp.float32),
    )(x, w1, degp)


def _combine_body(acc_ref, y_ref, degp_ref, b_ref, w_ref,
                  conv_ref, h_ref, ynext_ref):
    dinv = _dinv_from_deg(degp_ref)
    conv = dinv * (acc_ref[0] + acc_ref[1] + y_ref[...]) + b_ref[...]
    conv_ref[...] = conv
    h = jnp.maximum(conv, 0.0)
    h_ref[...] = h
    hw = lax.dot_general(h, w_ref[...], (((1,), (1,)), ((), ())),
                         preferred_element_type=jnp.float32)
    ynext_ref[...] = dinv * hw


def _combine_call(acc, y, degp, b, w_next):
    return pl.pallas_call(
        _combine_body,
        grid=(GRID,),
        in_specs=[
            pl.BlockSpec((NC, BLK, D), lambda j: (0, j, 0)),
            pl.BlockSpec((BLK, D), lambda j: (j, 0)),
            pl.BlockSpec((NC, BLK, 16), lambda j: (0, j, 0)),
            pl.BlockSpec((1, D), lambda j: (0, 0)),
            pl.BlockSpec((D, D), lambda j: (0, 0)),
        ],
        out_specs=[
            pl.BlockSpec((BLK, D), lambda j: (j, 0)),
            pl.BlockSpec((BLK, D), lambda j: (j, 0)),
            pl.BlockSpec((BLK, D), lambda j: (j, 0)),
        ],
        out_shape=[
            jax.ShapeDtypeStruct((NPAD, D), jnp.float32),
            jax.ShapeDtypeStruct((NPAD, D), jnp.float32),
            jax.ShapeDtypeStruct((NPAD, D), jnp.float32),
        ],
    )(acc, y, degp, b, w_next)


def _pool_body(acc_ref, y_ref, degp_ref, b_ref, batch_ref, wfc_ref, bfc_ref,
               conv_ref, pooled_ref, fc_ref, psum, pcnt):
    j = pl.program_id(0)
    dinv = _dinv_from_deg(degp_ref)
    conv = dinv * (acc_ref[0] + acc_ref[1] + y_ref[...]) + b_ref[...]
    conv_ref[...] = conv

    seg = lax.broadcasted_iota(jnp.int32, (1, G), 1)
    p = (batch_ref[...] == seg).astype(jnp.float32)
    part = lax.dot_general(p, conv, (((0,), (0,)), ((), ())),
                           preferred_element_type=jnp.float32)
    ones = jnp.ones((BLK, D), jnp.float32)
    cpart = lax.dot_general(p, ones, (((0,), (0,)), ((), ())),
                            preferred_element_type=jnp.float32)

    @pl.when(j == 0)
    def _():
        psum[...] = jnp.zeros((G, D), jnp.float32)
        pcnt[...] = jnp.zeros((G, D), jnp.float32)

    psum[...] += part
    pcnt[...] += cpart

    @pl.when(j == GRID - 1)
    def _():
        pooled = psum[...] / jnp.maximum(pcnt[...], 1.0)
        pooled_ref[...] = pooled
        fc = lax.dot_general(pooled, wfc_ref[...], (((1,), (1,)), ((), ())),
                             preferred_element_type=jnp.float32)
        fc_ref[...] = fc + bfc_ref[...]


def _pool_call(acc, y, degp, b, batch2d, wfc, bfc):
    ncls = wfc.shape[0]
    return pl.pallas_call(
        _pool_body,
        grid=(GRID,),
        in_specs=[
            pl.BlockSpec((NC, BLK, D), lambda j: (0, j, 0)),
            pl.BlockSpec((BLK, D), lambda j: (j, 0)),
            pl.BlockSpec((NC, BLK, 16), lambda j: (0, j, 0)),
            pl.BlockSpec((1, D), lambda j: (0, 0)),
            pl.BlockSpec((BLK, 1), lambda j: (j, 0)),
            pl.BlockSpec((ncls, D), lambda j: (0, 0)),
            pl.BlockSpec((1, ncls), lambda j: (0, 0)),
        ],
        out_specs=[
            pl.BlockSpec((BLK, D), lambda j: (j, 0)),
            pl.BlockSpec((G, D), lambda j: (0, 0)),
            pl.BlockSpec((G, ncls), lambda j: (0, 0)),
        ],
        out_shape=[
            jax.ShapeDtypeStruct((NPAD, D), jnp.float32),
            jax.ShapeDtypeStruct((G, D), jnp.float32),
            jax.ShapeDtypeStruct((G, ncls), jnp.float32),
        ],
        scratch_shapes=[
            pltpu.VMEM((G, D), jnp.float32),
            pltpu.VMEM((G, D), jnp.float32),
        ],
    )(acc, y, degp, b, batch2d, wfc, bfc)


def kernel(x, edge_index, batch, W1, b1, W2, b2, W3, b3, Wfc, bfc):
    src = edge_index[0].astype(jnp.int32)
    dst = edge_index[1].astype(jnp.int32)
    src_p = jnp.concatenate([src, jnp.zeros((EPAD - E,), jnp.int32)])
    dst_p = jnp.concatenate([dst, jnp.full((EPAD - E,), N, jnp.int32)])
    src_p = src_p.reshape(NW, NCH, CH)
    dst_p = dst_p.reshape(NW, NCH, CH)

    x_p = jnp.pad(x, ((0, NPAD - N), (0, 0)))
    batch_p = jnp.pad(batch.astype(jnp.int32), (0, NPAD - N),
                      constant_values=G).reshape(NPAD, 1)

    degp = _deg_kernel(dst_p)

    y1 = _y1_call(x_p, W1, degp)
    acc1 = _agg_kernel(y1, src_p, dst_p)
    conv1, relu1, y2 = _combine_call(acc1, y1, degp, b1.reshape(1, D), W2)

    acc2 = _agg_kernel(y2, src_p, dst_p)
    conv2, _relu2, y3 = _combine_call(acc2, y2, degp, b2.reshape(1, D), W3)

    acc3 = _agg_kernel(y3, src_p, dst_p)
    conv3, pooled, fc = _pool_call(acc3, y3, degp, b3.reshape(1, D),
                                   batch_p, Wfc, bfc.reshape(1, -1))

    activations = {
        "conv1": conv1[:N],
        "relu1": relu1[:N],
        "conv2": conv2[:N],
        "conv3": conv3[:N],
        "global_pool": pooled,
        "fc": fc,
    }
    return (fc, activations)
